# Initial kernel scaffold; baseline (speedup 1.0000x reference)
import jax
import jax.numpy as jnp
from jax.experimental import pallas as pl


def kernel(coord, atype, natoms, mapping, shift, selected, box, W0, b0, W1, b1, W2, b2, FW0, Fb0, FW1, Fb1, FW2, Fb2, bias_atom_e):
    raise NotImplementedError("write your pallas kernel here")


# speedup vs baseline: 1.8821x; 1.8821x over previous
import functools

import jax
import jax.numpy as jnp
from jax import lax
from jax.experimental import pallas as pl
from jax.experimental.pallas import tpu as pltpu
from jax.experimental.pallas import tpu_sc as plsc

NF = 1
NLOC = 10000
NALL = 12000
NNEI = 32
M = 64
AXIS = 8
NTYPES = 2

B = 64
P = B * NNEI
NLOCP = 10240
NBLK = NLOCP // B
NPF = NLOCP * NNEI
NP = NLOC * NNEI

TILES = 32
APT = NLOCP // TILES
FPAD = 30720
SL = FPAD // TILES
SH = 1200


def _tc_body(rx_r, ry_r, rz_r, at_r, W0T_r, b0c_r, W1_r, W1T_r, b1c_r, W2_r,
             W2T_r, b2c_r, FW0r_r, FW0rT_r, Fb0c_r, FW1_r, FW1T_r, Fb1c_r,
             FW2c_r, bias_r, gx_r, gy_r, gz_r, cs_r, acc_r):
    i = pl.program_id(0)
    f32 = jnp.float32

    pair_atom = jax.lax.broadcasted_iota(jnp.int32, (1, P), 1) // NNEI
    pmask = (pair_atom + i * B) < NLOC
    amask = (jax.lax.broadcasted_iota(jnp.int32, (1, B), 1) + i * B) < NLOC

    rx = jnp.where(pmask, rx_r[0], 1.0)
    ry = jnp.where(pmask, ry_r[0], 1.0)
    rz = jnp.where(pmask, rz_r[0], 1.0)

    rr = rx * rx + ry * ry + rz * rz + 1e-6
    inv_rr = 1.0 / rr
    s = 1.0 / jnp.sqrt(rr)


    def bl(x, rows):
        return jnp.broadcast_to(x, (rows, x.shape[1]))

    def bc(col, cols):
        return jnp.broadcast_to(col, (col.shape[0], cols))

    h0 = jnp.tanh(bc(W0T_r[...], P) * bl(s, 16) + bc(b0c_r[...], P))
    h1 = jnp.tanh(jnp.dot(W1T_r[...], h0, preferred_element_type=f32, precision=lax.Precision.HIGHEST)
                  + bc(b1c_r[...], P))
    g = jnp.tanh(jnp.dot(W2T_r[...], h1, preferred_element_type=f32, precision=lax.Precision.HIGHEST)
                 + bc(b2c_r[...], P))

    env = (s, rx / rr, ry / rr, rz / rr)

    pa = jax.lax.broadcasted_iota(jnp.int32, (P, B), 0) // NNEI
    nb = jax.lax.broadcasted_iota(jnp.int32, (P, B), 1)
    Seg = jnp.where(pa == nb, f32(1.0 / NNEI), f32(0.0))
    nb2 = jax.lax.broadcasted_iota(jnp.int32, (B, P), 0)
    pa2 = jax.lax.broadcasted_iota(jnp.int32, (B, P), 1) // NNEI
    E = jnp.where(nb2 == pa2, f32(1.0), f32(0.0))

    GR = [jnp.dot(g * bl(env[c], M), Seg, preferred_element_type=f32, precision=lax.Precision.HIGHEST)
          for c in range(4)]

    accf = bc(Fb0c_r[...], B)
    Da = []
    for a in range(AXIS):
        d_a = GR[0] * bl(GR[0][a:a + 1, :], M)
        for c in range(1, 4):
            d_a = d_a + GR[c] * bl(GR[c][a:a + 1, :], M)
        Da.append(d_a)
        accf = accf + jnp.dot(FW0rT_r[a], d_a, preferred_element_type=f32, precision=lax.Precision.HIGHEST)
    h0f = jnp.tanh(accf)
    h1f = jnp.tanh(jnp.dot(FW1T_r[...], h0f, preferred_element_type=f32, precision=lax.Precision.HIGHEST)
                   + bc(Fb1c_r[...], B))
    fw2b = bc(FW2c_r[...], B)
    aepre = jnp.sum(h1f * fw2b, axis=0, keepdims=True)
    bias0 = bias_r[0, 0]
    bias1 = bias_r[0, 1]
    atv = at_r[0]
    ae = aepre + jnp.where(atv == 0, bias0, bias1)
    esum = jnp.sum(jnp.where(amask, ae, 0.0))

    u1 = (1.0 - h1f * h1f) * fw2b
    u0 = jnp.dot(FW1_r[...], u1, preferred_element_type=f32, precision=lax.Precision.HIGHEST) * (1.0 - h0f * h0f)
    dGR = []
    dDa = [jnp.dot(FW0r_r[a], u0, preferred_element_type=f32, precision=lax.Precision.HIGHEST)
           for a in range(AXIS)]
    for c in range(4):
        t = dDa[0] * bl(GR[c][0:1, :], M)
        for a in range(1, AXIS):
            t = t + dDa[a] * bl(GR[c][a:a + 1, :], M)
        rows = [jnp.sum(dDa[m] * GR[c], axis=0, keepdims=True)
                for m in range(AXIS)]
        t2 = jnp.concatenate(rows + [jnp.zeros((M - AXIS, B), f32)], axis=0)
        dGR.append(t + t2)

    dGRe = [jnp.dot(dGR[c], E, preferred_element_type=f32, precision=lax.Precision.HIGHEST) for c in range(4)]
    dg = dGRe[0] * bl(env[0], M)
    for c in range(1, 4):
        dg = dg + dGRe[c] * bl(env[c], M)
    dg = dg * f32(1.0 / NNEI)
    denv = [jnp.sum(dGRe[c] * g, axis=0, keepdims=True) * f32(1.0 / NNEI)
            for c in range(4)]

    t2e = dg * (1.0 - g * g)
    dh1e = jnp.dot(W2_r[...], t2e, preferred_element_type=f32, precision=lax.Precision.HIGHEST) * (1.0 - h1 * h1)
    dh0e = jnp.dot(W1_r[...], dh1e, preferred_element_type=f32, precision=lax.Precision.HIGHEST) * (1.0 - h0 * h0)
    ds_embed = jnp.sum(dh0e * bc(W0T_r[...], P), axis=0, keepdims=True)

    ds_total = denv[0] + ds_embed
    dot = denv[1] * rx + denv[2] * ry + denv[3] * rz
    s3 = s * inv_rr
    common = 2.0 * dot * inv_rr * inv_rr
    gx = denv[1] * inv_rr - rx * common - ds_total * rx * s3
    gy = denv[2] * inv_rr - ry * common - ds_total * ry * s3
    gz = denv[3] * inv_rr - rz * common - ds_total * rz * s3
    gx = jnp.where(pmask, gx, 0.0)
    gy = jnp.where(pmask, gy, 0.0)
    gz = jnp.where(pmask, gz, 0.0)
    gx_r[...] = gx[None]
    gy_r[...] = gy[None]
    gz_r[...] = gz[None]

    csx = jnp.dot(gx, Seg, preferred_element_type=f32, precision=lax.Precision.HIGHEST) * f32(NNEI)
    csy = jnp.dot(gy, Seg, preferred_element_type=f32, precision=lax.Precision.HIGHEST) * f32(NNEI)
    csz = jnp.dot(gz, Seg, preferred_element_type=f32, precision=lax.Precision.HIGHEST) * f32(NNEI)
    cs_r[...] = jnp.concatenate([csx, csy, csz], axis=1)[None]

    parts = [esum]
    for rc in (rx, ry, rz):
        for gc in (gx, gy, gz):
            parts.append(-jnp.sum(rc * gc))
    pvec = jnp.concatenate([x.reshape(1, 1) for x in parts], axis=1)
    pvec = jnp.concatenate([pvec, jnp.zeros((1, 128 - len(parts)), f32)],
                           axis=1)

    @pl.when(i == 0)
    def _():
        acc_r[...] = jnp.zeros_like(acc_r)

    acc_r[...] += pvec


def _tc_call(rx, ry, rz, atype_p, W0T, b0c, W1, W1T, b1c, W2, W2T, b2c, FW0r,
             FW0rT, Fb0c, FW1, FW1T, Fb1c, FW2c, biasadj):
    f32 = jnp.float32
    full = lambda shp: pl.BlockSpec(shp, lambda i: (0,) * len(shp))
    row = pl.BlockSpec((1, 1, P), lambda i: (i, 0, 0))
    out = pl.pallas_call(
        _tc_body,
        grid=(NBLK,),
        in_specs=[
            row, row, row,
            pl.BlockSpec((1, 1, B), lambda i: (i, 0, 0)),
            full((16, 1)), full((16, 1)), full((16, 32)), full((32, 16)),
            full((32, 1)), full((32, 64)), full((64, 32)), full((64, 1)),
            full((8, 64, 128)), full((8, 128, 64)), full((128, 1)),
            full((128, 128)), full((128, 128)), full((128, 1)),
            full((128, 1)), full((1, NTYPES)),
        ],
        out_specs=[
            row, row, row,
            pl.BlockSpec((1, 1, 192), lambda i: (i, 0, 0)),
            pl.BlockSpec((1, 128), lambda i: (0, 0)),
        ],
        out_shape=[
            jax.ShapeDtypeStruct((NBLK, 1, P), f32),
            jax.ShapeDtypeStruct((NBLK, 1, P), f32),
            jax.ShapeDtypeStruct((NBLK, 1, P), f32),
            jax.ShapeDtypeStruct((NBLK, 1, 192), f32),
            jax.ShapeDtypeStruct((1, 128), f32),
        ],
        compiler_params=pltpu.CompilerParams(
            dimension_semantics=("arbitrary",)),
    )(rx, ry, rz, atype_p, W0T, b0c, W1, W1T, b1c, W2, W2T, b2c, FW0r, FW0rT,
      Fb0c, FW1, FW1T, Fb1c, FW2c, biasadj)
    return out


def _sc_gather_body(cx_h, cy_h, cz_h, map_h, shx_h, shy_h, shz_h, sel_h,
                    rx_h, ry_h, rz_h,
                    cx, cy, cz, mp, ecx, ecy, ecz, sx, sy, sz, selv,
                    ox, oy, oz):
    info = plsc.get_sparse_core_info()
    wid = lax.axis_index("s") * info.num_cores + lax.axis_index("c")

    pltpu.sync_copy(cx_h, cx)
    pltpu.sync_copy(cy_h, cy)
    pltpu.sync_copy(cz_h, cz)
    pltpu.sync_copy(map_h, mp)

    def chunk_body(ci, carry):
        base = ci * SH
        pltpu.sync_copy(shx_h.at[pl.ds(base, SH)], sx)
        pltpu.sync_copy(shy_h.at[pl.ds(base, SH)], sy)
        pltpu.sync_copy(shz_h.at[pl.ds(base, SH)], sz)

        def vec_body(vi, c2):
            o = vi * 16
            idx = mp[pl.ds(base + o, 16)]
            ecx[pl.ds(base + o, 16)] = plsc.load_gather(cx, [idx]) - sx[pl.ds(o, 16)]
            ecy[pl.ds(base + o, 16)] = plsc.load_gather(cy, [idx]) - sy[pl.ds(o, 16)]
            ecz[pl.ds(base + o, 16)] = plsc.load_gather(cz, [idx]) - sz[pl.ds(o, 16)]
            return c2

        return lax.fori_loop(0, SH // 16, vec_body, carry)

    lax.fori_loop(0, NALL // SH, chunk_body, 0)

    a0 = wid * APT
    pltpu.sync_copy(sel_h.at[pl.ds(a0 * NNEI, APT * NNEI)], selv)

    CH = 16

    def out_chunk(oi, carry):
        def atom_body(ai, c2):
            i = oi * CH + ai
            n = a0 + i
            nv = jnp.full((16,), n, jnp.int32)
            cx0 = plsc.load_gather(ecx, [nv])
            cy0 = plsc.load_gather(ecy, [nv])
            cz0 = plsc.load_gather(ecz, [nv])
            for u in range(NNEI // 16):
                jo = i * NNEI + u * 16
                so = ai * NNEI + u * 16
                jdx = selv[pl.ds(jo, 16)]
                ox[pl.ds(so, 16)] = plsc.load_gather(ecx, [jdx]) - cx0
                oy[pl.ds(so, 16)] = plsc.load_gather(ecy, [jdx]) - cy0
                oz[pl.ds(so, 16)] = plsc.load_gather(ecz, [jdx]) - cz0
            return c2

        lax.fori_loop(0, CH, atom_body, 0)
        dst = pl.ds((a0 + oi * CH) * NNEI, CH * NNEI)
        pltpu.sync_copy(ox, rx_h.at[dst])
        pltpu.sync_copy(oy, ry_h.at[dst])
        pltpu.sync_copy(oz, rz_h.at[dst])
        return carry

    lax.fori_loop(0, APT // CH, out_chunk, 0)


def _sc_gather(cx, cy, cz, mapping, shx, shy, shz, sel_pad):
    f32 = jnp.float32
    mesh = plsc.VectorSubcoreMesh(core_axis_name="c", subcore_axis_name="s")
    kern = pl.kernel(
        _sc_gather_body,
        out_type=[jax.ShapeDtypeStruct((NPF,), f32)] * 3,
        mesh=mesh,
        scratch_types=[
            pltpu.VMEM((NLOC,), f32),
            pltpu.VMEM((NLOC,), f32),
            pltpu.VMEM((NLOC,), f32),
            pltpu.VMEM((NALL,), jnp.int32),
            pltpu.VMEM((NALL,), f32),
            pltpu.VMEM((NALL,), f32),
            pltpu.VMEM((NALL,), f32),
            pltpu.VMEM((SH,), f32),
            pltpu.VMEM((SH,), f32),
            pltpu.VMEM((SH,), f32),
            pltpu.VMEM((APT * NNEI,), jnp.int32),
            pltpu.VMEM((16 * NNEI,), f32),
            pltpu.VMEM((16 * NNEI,), f32),
            pltpu.VMEM((16 * NNEI,), f32),
        ],
        compiler_params=pltpu.CompilerParams(needs_layout_passes=False),
    )
    return kern(cx, cy, cz, mapping, shx, shy, shz, sel_pad)


def _sc_scatter_body(gx_h, gy_h, gz_h, cs_h, map_h, sel_h, part_h,
                     facc, mp, selv, gxv, gyv, gzv, csv):
    info = plsc.get_sparse_core_info()
    wid = lax.axis_index("s") * info.num_cores + lax.axis_index("c")
    a0 = wid * APT

    pltpu.sync_copy(map_h, mp)
    pltpu.sync_copy(sel_h.at[pl.ds(a0 * NNEI, APT * NNEI)], selv)
    pltpu.sync_copy(gx_h.at[pl.ds(a0 * NNEI, APT * NNEI)], gxv)
    pltpu.sync_copy(gy_h.at[pl.ds(a0 * NNEI, APT * NNEI)], gyv)
    pltpu.sync_copy(gz_h.at[pl.ds(a0 * NNEI, APT * NNEI)], gzv)
    pltpu.sync_copy(cs_h.at[pl.ds((a0 // B) * 192, (APT // B) * 192)], csv)

    zero16 = jnp.zeros((16,), jnp.float32)

    def zero_body(i, c):
        facc[pl.ds(i * 16, 16)] = zero16
        return c

    lax.fori_loop(0, FPAD // 16, zero_body, 0)

    nvalid = jnp.maximum(0, jnp.minimum(APT, NLOC - a0))

    def scatter3(midx, vx, vy, vz):
        cnt, _ = plsc.scan_count(midx)
        maxc = jnp.max(cnt, axis=0)
        t = midx * 3

        def round_body(r, c):
            m = cnt == r
            plsc.addupdate_scatter(facc, [t], vx, mask=m)
            plsc.addupdate_scatter(facc, [t + 1], vy, mask=m)
            plsc.addupdate_scatter(facc, [t + 2], vz, mask=m)
            return c

        return lax.fori_loop(1, maxc + 1, round_body, 0)

    def atom_body(i, c):
        for u in range(NNEI // 16):
            off = i * NNEI + u * 16
            jdx = selv[pl.ds(off, 16)]
            msel = plsc.load_gather(mp, [jdx])
            scatter3(msel, -gxv[pl.ds(off, 16)], -gyv[pl.ds(off, 16)],
                     -gzv[pl.ds(off, 16)])
        return c

    lax.fori_loop(0, nvalid, atom_body, 0)

    def cent_body(k, c):
        i = k * 16
        mvec = mp[pl.ds(a0 + i, 16)]
        blk = (i // B) * 192
        io = i % B
        scatter3(mvec, csv[pl.ds(blk + io, 16)], csv[pl.ds(blk + B + io, 16)],
                 csv[pl.ds(blk + 2 * B + io, 16)])
        return c

    lax.fori_loop(0, nvalid // 16, cent_body, 0)

    pltpu.sync_copy(facc, part_h.at[pl.ds(wid * FPAD, FPAD)])


def _sc_scatter(gx, gy, gz, cs, mapping, sel_pad):
    f32 = jnp.float32
    mesh = plsc.VectorSubcoreMesh(core_axis_name="c", subcore_axis_name="s")
    kern = pl.kernel(
        _sc_scatter_body,
        out_type=jax.ShapeDtypeStruct((TILES * FPAD,), f32),
        mesh=mesh,
        scratch_types=[
            pltpu.VMEM((FPAD,), f32),
            pltpu.VMEM((NALL,), jnp.int32),
            pltpu.VMEM((APT * NNEI,), jnp.int32),
            pltpu.VMEM((APT * NNEI,), f32),
            pltpu.VMEM((APT * NNEI,), f32),
            pltpu.VMEM((APT * NNEI,), f32),
            pltpu.VMEM(((APT // B) * 192,), f32),
        ],
        compiler_params=pltpu.CompilerParams(needs_layout_passes=False),
    )
    return kern(gx, gy, gz, cs, mapping, sel_pad)


def _sc_reduce_body(part_h, force_h, pbuf, obuf):
    info = plsc.get_sparse_core_info()
    wid = lax.axis_index("s") * info.num_cores + lax.axis_index("c")
    base = wid * SL
    for t in range(TILES):
        pltpu.sync_copy(part_h.at[pl.ds(t * FPAD + base, SL)],
                        pbuf.at[pl.ds(t * SL, SL)])

    def vec_body(v, c):
        o = v * 16
        acc = pbuf[pl.ds(o, 16)]
        for t in range(1, TILES):
            acc = acc + pbuf[pl.ds(t * SL + o, 16)]
        obuf[pl.ds(o, 16)] = acc
        return c

    lax.fori_loop(0, SL // 16, vec_body, 0)
    pltpu.sync_copy(obuf, force_h.at[pl.ds(base, SL)])


def _sc_reduce(partials):
    f32 = jnp.float32
    mesh = plsc.VectorSubcoreMesh(core_axis_name="c", subcore_axis_name="s")
    kern = pl.kernel(
        _sc_reduce_body,
        out_type=jax.ShapeDtypeStruct((FPAD,), f32),
        mesh=mesh,
        scratch_types=[
            pltpu.VMEM((TILES * SL,), f32),
            pltpu.VMEM((SL,), f32),
        ],
        compiler_params=pltpu.CompilerParams(needs_layout_passes=False),
    )
    return kern(partials)


def kernel(coord, atype, natoms, mapping, shift, selected, box, W0, b0, W1,
           b1, W2, b2, FW0, Fb0, FW1, Fb1, FW2, Fb2, bias_atom_e):
    f32 = jnp.float32
    cx = coord[0, :, 0]
    cy = coord[0, :, 1]
    cz = coord[0, :, 2]
    shx = shift[0, :, 0]
    shy = shift[0, :, 1]
    shz = shift[0, :, 2]
    map0 = mapping[0].astype(jnp.int32)
    sel_pad = jnp.concatenate(
        [selected[0].reshape(NP).astype(jnp.int32),
         jnp.zeros((NPF - NP,), jnp.int32)])

    rx, ry, rz = _sc_gather(cx, cy, cz, map0, shx, shy, shz, sel_pad)
    rx = rx.reshape(NBLK, 1, P)
    ry = ry.reshape(NBLK, 1, P)
    rz = rz.reshape(NBLK, 1, P)

    atype_p = jnp.concatenate(
        [atype[0].astype(jnp.int32), jnp.zeros((NLOCP - NLOC,), jnp.int32)]
    ).reshape(NBLK, 1, B)
    W0T = W0.T
    b0c = b0.reshape(16, 1)
    W1T = W1.T
    b1c = b1.reshape(32, 1)
    W2T = W2.T
    b2c = b2.reshape(M, 1)
    FW0r = FW0.reshape(M, AXIS, 128).transpose(1, 0, 2)
    FW0rT = FW0r.transpose(0, 2, 1)
    Fb0c = Fb0.reshape(128, 1)
    FW1T = FW1.T
    Fb1c = Fb1.reshape(128, 1)
    biasadj = (bias_atom_e + Fb2[0]).reshape(1, NTYPES)

    gx, gy, gz, cs, acc = _tc_call(rx, ry, rz, atype_p, W0T, b0c, W1, W1T,
                                   b1c, W2, W2T, b2c, FW0r, FW0rT, Fb0c, FW1,
                                   FW1T, Fb1c, FW2, biasadj)

    partials = _sc_scatter(gx.reshape(NPF), gy.reshape(NPF), gz.reshape(NPF),
                           cs.reshape(NBLK * 192), map0, sel_pad)
    force_flat = _sc_reduce(partials)

    energy = acc[0, 0:1]
    virial = acc[0, 1:10].reshape(1, 3, 3)
    force = force_flat[:NLOC * 3].reshape(1, NLOC, 3)
    return energy, force, virial


# speedup vs baseline: 1.8857x; 1.0019x over previous
import functools

import jax
import jax.numpy as jnp
from jax import lax
from jax.experimental import pallas as pl
from jax.experimental.pallas import tpu as pltpu
from jax.experimental.pallas import tpu_sc as plsc

NF = 1
NLOC = 10000
NALL = 12000
NNEI = 32
M = 64
AXIS = 8
NTYPES = 2

B = 64
P = B * NNEI
NLOCP = 10240
NBLK = NLOCP // B
NPF = NLOCP * NNEI
NP = NLOC * NNEI

TILES = 32
APT = NLOCP // TILES
FPAD = 30720
SL = FPAD // TILES
SH = 1200


def _tc_body(rx_r, ry_r, rz_r, at_r, W0T_r, b0c_r, W1_r, W1T_r, b1c_r, W2_r,
             W2T_r, b2c_r, FW0r_r, FW0rT_r, Fb0c_r, FW1_r, FW1T_r, Fb1c_r,
             FW2c_r, bias_r, Seg_r, E_r, gx_r, gy_r, gz_r, cs_r, acc_r):
    i = pl.program_id(0)
    f32 = jnp.float32

    pair_atom = jax.lax.broadcasted_iota(jnp.int32, (1, P), 1) // NNEI
    pmask = (pair_atom + i * B) < NLOC
    amask = (jax.lax.broadcasted_iota(jnp.int32, (1, B), 1) + i * B) < NLOC

    rx = jnp.where(pmask, rx_r[0], 1.0)
    ry = jnp.where(pmask, ry_r[0], 1.0)
    rz = jnp.where(pmask, rz_r[0], 1.0)

    rr = rx * rx + ry * ry + rz * rz + 1e-6
    inv_rr = 1.0 / rr
    s = 1.0 / jnp.sqrt(rr)


    def bl(x, rows):
        return jnp.broadcast_to(x, (rows, x.shape[1]))

    def bc(col, cols):
        return jnp.broadcast_to(col, (col.shape[0], cols))

    h0 = jnp.tanh(bc(W0T_r[...], P) * bl(s, 16) + bc(b0c_r[...], P))
    h1 = jnp.tanh(jnp.dot(W1T_r[...], h0, preferred_element_type=f32, precision=lax.Precision.HIGHEST)
                  + bc(b1c_r[...], P))
    g = jnp.tanh(jnp.dot(W2T_r[...], h1, preferred_element_type=f32, precision=lax.Precision.HIGHEST)
                 + bc(b2c_r[...], P))

    env = (s, rx / rr, ry / rr, rz / rr)

    Seg = Seg_r[...]
    E = E_r[...]

    GR = [jnp.dot(g * bl(env[c], M), Seg, preferred_element_type=f32, precision=lax.Precision.HIGHEST)
          for c in range(4)]

    accf = bc(Fb0c_r[...], B)
    for a in range(AXIS):
        d_a = GR[0] * bl(GR[0][a:a + 1, :], M)
        for c in range(1, 4):
            d_a = d_a + GR[c] * bl(GR[c][a:a + 1, :], M)
        accf = accf + jnp.dot(FW0rT_r[a], d_a, preferred_element_type=f32, precision=lax.Precision.HIGHEST)
    h0f = jnp.tanh(accf)
    h1f = jnp.tanh(jnp.dot(FW1T_r[...], h0f, preferred_element_type=f32, precision=lax.Precision.HIGHEST)
                   + bc(Fb1c_r[...], B))
    fw2b = bc(FW2c_r[...], B)
    aepre = jnp.sum(h1f * fw2b, axis=0, keepdims=True)
    bias0 = bias_r[0, 0]
    bias1 = bias_r[0, 1]
    atv = at_r[0]
    ae = aepre + jnp.where(atv == 0, bias0, bias1)
    esum = jnp.sum(jnp.where(amask, ae, 0.0))

    u1 = (1.0 - h1f * h1f) * fw2b
    u0 = jnp.dot(FW1_r[...], u1, preferred_element_type=f32, precision=lax.Precision.HIGHEST) * (1.0 - h0f * h0f)
    dGR = []
    dDa = [jnp.dot(FW0r_r[a], u0, preferred_element_type=f32, precision=lax.Precision.HIGHEST)
           for a in range(AXIS)]
    for c in range(4):
        t = dDa[0] * bl(GR[c][0:1, :], M)
        for a in range(1, AXIS):
            t = t + dDa[a] * bl(GR[c][a:a + 1, :], M)
        rows = [jnp.sum(dDa[m] * GR[c], axis=0, keepdims=True)
                for m in range(AXIS)]
        t2 = jnp.concatenate(rows + [jnp.zeros((M - AXIS, B), f32)], axis=0)
        dGR.append(t + t2)

    dGRe = [jnp.dot(dGR[c], E, preferred_element_type=f32, precision=lax.Precision.HIGHEST) for c in range(4)]
    dg = dGRe[0] * bl(env[0], M)
    for c in range(1, 4):
        dg = dg + dGRe[c] * bl(env[c], M)
    denv = [jnp.sum(dGRe[c] * g, axis=0, keepdims=True)
            for c in range(4)]

    t2e = dg * (1.0 - g * g)
    dh1e = jnp.dot(W2_r[...], t2e, preferred_element_type=f32, precision=lax.Precision.HIGHEST) * (1.0 - h1 * h1)
    dh0e = jnp.dot(W1_r[...], dh1e, preferred_element_type=f32, precision=lax.Precision.HIGHEST) * (1.0 - h0 * h0)
    ds_embed = jnp.sum(dh0e * bc(W0T_r[...], P), axis=0, keepdims=True)

    ds_total = denv[0] + ds_embed
    dot = denv[1] * rx + denv[2] * ry + denv[3] * rz
    s3 = s * inv_rr
    common = 2.0 * dot * inv_rr * inv_rr
    gx = denv[1] * inv_rr - rx * common - ds_total * rx * s3
    gy = denv[2] * inv_rr - ry * common - ds_total * ry * s3
    gz = denv[3] * inv_rr - rz * common - ds_total * rz * s3
    gx = jnp.where(pmask, gx, 0.0)
    gy = jnp.where(pmask, gy, 0.0)
    gz = jnp.where(pmask, gz, 0.0)
    gx_r[...] = gx[None]
    gy_r[...] = gy[None]
    gz_r[...] = gz[None]

    csx = jnp.dot(gx, Seg, preferred_element_type=f32, precision=lax.Precision.HIGHEST) * f32(NNEI)
    csy = jnp.dot(gy, Seg, preferred_element_type=f32, precision=lax.Precision.HIGHEST) * f32(NNEI)
    csz = jnp.dot(gz, Seg, preferred_element_type=f32, precision=lax.Precision.HIGHEST) * f32(NNEI)
    cs_r[...] = jnp.concatenate([csx, csy, csz], axis=1)[None]

    parts = [esum]
    for rc in (rx, ry, rz):
        for gc in (gx, gy, gz):
            parts.append(-jnp.sum(rc * gc))
    pvec = jnp.concatenate([x.reshape(1, 1) for x in parts], axis=1)
    pvec = jnp.concatenate([pvec, jnp.zeros((1, 128 - len(parts)), f32)],
                           axis=1)

    @pl.when(i == 0)
    def _():
        acc_r[...] = jnp.zeros_like(acc_r)

    acc_r[...] += pvec


def _tc_call(rx, ry, rz, atype_p, W0T, b0c, W1, W1T, b1c, W2, W2T, b2c, FW0r,
             FW0rT, Fb0c, FW1, FW1T, Fb1c, FW2c, biasadj, Seg, E):
    f32 = jnp.float32
    full = lambda shp: pl.BlockSpec(shp, lambda i: (0,) * len(shp))
    row = pl.BlockSpec((1, 1, P), lambda i: (i, 0, 0))
    out = pl.pallas_call(
        _tc_body,
        grid=(NBLK,),
        in_specs=[
            row, row, row,
            pl.BlockSpec((1, 1, B), lambda i: (i, 0, 0)),
            full((16, 1)), full((16, 1)), full((16, 32)), full((32, 16)),
            full((32, 1)), full((32, 64)), full((64, 32)), full((64, 1)),
            full((8, 64, 128)), full((8, 128, 64)), full((128, 1)),
            full((128, 128)), full((128, 128)), full((128, 1)),
            full((128, 1)), full((1, NTYPES)),
            full((P, B)), full((B, P)),
        ],
        out_specs=[
            row, row, row,
            pl.BlockSpec((1, 1, 192), lambda i: (i, 0, 0)),
            pl.BlockSpec((1, 128), lambda i: (0, 0)),
        ],
        out_shape=[
            jax.ShapeDtypeStruct((NBLK, 1, P), f32),
            jax.ShapeDtypeStruct((NBLK, 1, P), f32),
            jax.ShapeDtypeStruct((NBLK, 1, P), f32),
            jax.ShapeDtypeStruct((NBLK, 1, 192), f32),
            jax.ShapeDtypeStruct((1, 128), f32),
        ],
        compiler_params=pltpu.CompilerParams(
            dimension_semantics=("arbitrary",)),
    )(rx, ry, rz, atype_p, W0T, b0c, W1, W1T, b1c, W2, W2T, b2c, FW0r, FW0rT,
      Fb0c, FW1, FW1T, Fb1c, FW2c, biasadj, Seg, E)
    return out


def _sc_gather_body(cx_h, cy_h, cz_h, map_h, shx_h, shy_h, shz_h, sel_h,
                    rx_h, ry_h, rz_h,
                    cx, cy, cz, mp, ecx, ecy, ecz, sx, sy, sz, selv,
                    ox, oy, oz):
    info = plsc.get_sparse_core_info()
    wid = lax.axis_index("s") * info.num_cores + lax.axis_index("c")

    pltpu.sync_copy(cx_h, cx)
    pltpu.sync_copy(cy_h, cy)
    pltpu.sync_copy(cz_h, cz)
    pltpu.sync_copy(map_h, mp)

    def chunk_body(ci, carry):
        base = ci * SH
        pltpu.sync_copy(shx_h.at[pl.ds(base, SH)], sx)
        pltpu.sync_copy(shy_h.at[pl.ds(base, SH)], sy)
        pltpu.sync_copy(shz_h.at[pl.ds(base, SH)], sz)

        def vec_body(vi, c2):
            o = vi * 16
            idx = mp[pl.ds(base + o, 16)]
            ecx[pl.ds(base + o, 16)] = plsc.load_gather(cx, [idx]) - sx[pl.ds(o, 16)]
            ecy[pl.ds(base + o, 16)] = plsc.load_gather(cy, [idx]) - sy[pl.ds(o, 16)]
            ecz[pl.ds(base + o, 16)] = plsc.load_gather(cz, [idx]) - sz[pl.ds(o, 16)]
            return c2

        return lax.fori_loop(0, SH // 16, vec_body, carry)

    lax.fori_loop(0, NALL // SH, chunk_body, 0)

    a0 = wid * APT
    pltpu.sync_copy(sel_h.at[pl.ds(a0 * NNEI, APT * NNEI)], selv)

    CH = 16

    def out_chunk(oi, carry):
        def atom_body(ai, c2):
            i = oi * CH + ai
            n = a0 + i
            nv = jnp.full((16,), n, jnp.int32)
            cx0 = plsc.load_gather(ecx, [nv])
            cy0 = plsc.load_gather(ecy, [nv])
            cz0 = plsc.load_gather(ecz, [nv])
            for u in range(NNEI // 16):
                jo = i * NNEI + u * 16
                so = ai * NNEI + u * 16
                jdx = selv[pl.ds(jo, 16)]
                ox[pl.ds(so, 16)] = plsc.load_gather(ecx, [jdx]) - cx0
                oy[pl.ds(so, 16)] = plsc.load_gather(ecy, [jdx]) - cy0
                oz[pl.ds(so, 16)] = plsc.load_gather(ecz, [jdx]) - cz0
            return c2

        lax.fori_loop(0, CH, atom_body, 0)
        dst = pl.ds((a0 + oi * CH) * NNEI, CH * NNEI)
        pltpu.sync_copy(ox, rx_h.at[dst])
        pltpu.sync_copy(oy, ry_h.at[dst])
        pltpu.sync_copy(oz, rz_h.at[dst])
        return carry

    lax.fori_loop(0, APT // CH, out_chunk, 0)


def _sc_gather(cx, cy, cz, mapping, shx, shy, shz, sel_pad):
    f32 = jnp.float32
    mesh = plsc.VectorSubcoreMesh(core_axis_name="c", subcore_axis_name="s")
    kern = pl.kernel(
        _sc_gather_body,
        out_type=[jax.ShapeDtypeStruct((NPF,), f32)] * 3,
        mesh=mesh,
        scratch_types=[
            pltpu.VMEM((NLOC,), f32),
            pltpu.VMEM((NLOC,), f32),
            pltpu.VMEM((NLOC,), f32),
            pltpu.VMEM((NALL,), jnp.int32),
            pltpu.VMEM((NALL,), f32),
            pltpu.VMEM((NALL,), f32),
            pltpu.VMEM((NALL,), f32),
            pltpu.VMEM((SH,), f32),
            pltpu.VMEM((SH,), f32),
            pltpu.VMEM((SH,), f32),
            pltpu.VMEM((APT * NNEI,), jnp.int32),
            pltpu.VMEM((16 * NNEI,), f32),
            pltpu.VMEM((16 * NNEI,), f32),
            pltpu.VMEM((16 * NNEI,), f32),
        ],
        compiler_params=pltpu.CompilerParams(needs_layout_passes=False),
    )
    return kern(cx, cy, cz, mapping, shx, shy, shz, sel_pad)


def _sc_scatter_body(gx_h, gy_h, gz_h, cs_h, map_h, sel_h, part_h,
                     facc, mp, selv, gxv, gyv, gzv, csv):
    info = plsc.get_sparse_core_info()
    wid = lax.axis_index("s") * info.num_cores + lax.axis_index("c")
    a0 = wid * APT

    pltpu.sync_copy(map_h, mp)
    pltpu.sync_copy(sel_h.at[pl.ds(a0 * NNEI, APT * NNEI)], selv)
    pltpu.sync_copy(gx_h.at[pl.ds(a0 * NNEI, APT * NNEI)], gxv)
    pltpu.sync_copy(gy_h.at[pl.ds(a0 * NNEI, APT * NNEI)], gyv)
    pltpu.sync_copy(gz_h.at[pl.ds(a0 * NNEI, APT * NNEI)], gzv)
    pltpu.sync_copy(cs_h.at[pl.ds((a0 // B) * 192, (APT // B) * 192)], csv)

    zero16 = jnp.zeros((16,), jnp.float32)

    def zero_body(i, c):
        facc[pl.ds(i * 16, 16)] = zero16
        return c

    lax.fori_loop(0, FPAD // 16, zero_body, 0)

    nvalid = jnp.maximum(0, jnp.minimum(APT, NLOC - a0))

    def scatter3(midx, vx, vy, vz):
        cnt, _ = plsc.scan_count(midx)
        maxc = jnp.max(cnt, axis=0)
        t = midx * 3

        def round_body(r, c):
            m = cnt == r
            plsc.addupdate_scatter(facc, [t], vx, mask=m)
            plsc.addupdate_scatter(facc, [t + 1], vy, mask=m)
            plsc.addupdate_scatter(facc, [t + 2], vz, mask=m)
            return c

        return lax.fori_loop(1, maxc + 1, round_body, 0)

    def atom_body(i, c):
        for u in range(NNEI // 16):
            off = i * NNEI + u * 16
            jdx = selv[pl.ds(off, 16)]
            msel = plsc.load_gather(mp, [jdx])
            scatter3(msel, -gxv[pl.ds(off, 16)], -gyv[pl.ds(off, 16)],
                     -gzv[pl.ds(off, 16)])
        return c

    lax.fori_loop(0, nvalid, atom_body, 0)

    def cent_body(k, c):
        i = k * 16
        mvec = mp[pl.ds(a0 + i, 16)]
        blk = (i // B) * 192
        io = i % B
        scatter3(mvec, csv[pl.ds(blk + io, 16)], csv[pl.ds(blk + B + io, 16)],
                 csv[pl.ds(blk + 2 * B + io, 16)])
        return c

    lax.fori_loop(0, nvalid // 16, cent_body, 0)

    pltpu.sync_copy(facc, part_h.at[pl.ds(wid * FPAD, FPAD)])


def _sc_scatter(gx, gy, gz, cs, mapping, sel_pad):
    f32 = jnp.float32
    mesh = plsc.VectorSubcoreMesh(core_axis_name="c", subcore_axis_name="s")
    kern = pl.kernel(
        _sc_scatter_body,
        out_type=jax.ShapeDtypeStruct((TILES * FPAD,), f32),
        mesh=mesh,
        scratch_types=[
            pltpu.VMEM((FPAD,), f32),
            pltpu.VMEM((NALL,), jnp.int32),
            pltpu.VMEM((APT * NNEI,), jnp.int32),
            pltpu.VMEM((APT * NNEI,), f32),
            pltpu.VMEM((APT * NNEI,), f32),
            pltpu.VMEM((APT * NNEI,), f32),
            pltpu.VMEM(((APT // B) * 192,), f32),
        ],
        compiler_params=pltpu.CompilerParams(needs_layout_passes=False),
    )
    return kern(gx, gy, gz, cs, mapping, sel_pad)


def _sc_reduce_body(part_h, force_h, pbuf, obuf):
    info = plsc.get_sparse_core_info()
    wid = lax.axis_index("s") * info.num_cores + lax.axis_index("c")
    base = wid * SL
    for t in range(TILES):
        pltpu.sync_copy(part_h.at[pl.ds(t * FPAD + base, SL)],
                        pbuf.at[pl.ds(t * SL, SL)])

    def vec_body(v, c):
        o = v * 16
        acc = pbuf[pl.ds(o, 16)]
        for t in range(1, TILES):
            acc = acc + pbuf[pl.ds(t * SL + o, 16)]
        obuf[pl.ds(o, 16)] = acc
        return c

    lax.fori_loop(0, SL // 16, vec_body, 0)
    pltpu.sync_copy(obuf, force_h.at[pl.ds(base, SL)])


def _sc_reduce(partials):
    f32 = jnp.float32
    mesh = plsc.VectorSubcoreMesh(core_axis_name="c", subcore_axis_name="s")
    kern = pl.kernel(
        _sc_reduce_body,
        out_type=jax.ShapeDtypeStruct((FPAD,), f32),
        mesh=mesh,
        scratch_types=[
            pltpu.VMEM((TILES * SL,), f32),
            pltpu.VMEM((SL,), f32),
        ],
        compiler_params=pltpu.CompilerParams(needs_layout_passes=False),
    )
    return kern(partials)


def kernel(coord, atype, natoms, mapping, shift, selected, box, W0, b0, W1,
           b1, W2, b2, FW0, Fb0, FW1, Fb1, FW2, Fb2, bias_atom_e):
    f32 = jnp.float32
    cx = coord[0, :, 0]
    cy = coord[0, :, 1]
    cz = coord[0, :, 2]
    shx = shift[0, :, 0]
    shy = shift[0, :, 1]
    shz = shift[0, :, 2]
    map0 = mapping[0].astype(jnp.int32)
    sel_pad = jnp.concatenate(
        [selected[0].reshape(NP).astype(jnp.int32),
         jnp.zeros((NPF - NP,), jnp.int32)])

    rx, ry, rz = _sc_gather(cx, cy, cz, map0, shx, shy, shz, sel_pad)
    rx = rx.reshape(NBLK, 1, P)
    ry = ry.reshape(NBLK, 1, P)
    rz = rz.reshape(NBLK, 1, P)

    atype_p = jnp.concatenate(
        [atype[0].astype(jnp.int32), jnp.zeros((NLOCP - NLOC,), jnp.int32)]
    ).reshape(NBLK, 1, B)
    W0T = W0.T
    b0c = b0.reshape(16, 1)
    W1T = W1.T
    b1c = b1.reshape(32, 1)
    W2T = W2.T
    b2c = b2.reshape(M, 1)
    FW0r = FW0.reshape(M, AXIS, 128).transpose(1, 0, 2)
    FW0rT = FW0r.transpose(0, 2, 1)
    Fb0c = Fb0.reshape(128, 1)
    FW1T = FW1.T
    Fb1c = Fb1.reshape(128, 1)
    biasadj = (bias_atom_e + Fb2[0]).reshape(1, NTYPES)
    pa = jnp.arange(P, dtype=jnp.int32)[:, None] // NNEI
    nb = jnp.arange(B, dtype=jnp.int32)[None, :]
    Seg = jnp.where(pa == nb, jnp.float32(1.0 / NNEI), 0.0)
    E = Seg.T

    gx, gy, gz, cs, acc = _tc_call(rx, ry, rz, atype_p, W0T, b0c, W1, W1T,
                                   b1c, W2, W2T, b2c, FW0r, FW0rT, Fb0c, FW1,
                                   FW1T, Fb1c, FW2, biasadj, Seg, E)

    partials = _sc_scatter(gx.reshape(NPF), gy.reshape(NPF), gz.reshape(NPF),
                           cs.reshape(NBLK * 192), map0, sel_pad)
    force_flat = _sc_reduce(partials)

    energy = acc[0, 0:1]
    virial = acc[0, 1:10].reshape(1, 3, 3)
    force = force_flat[:NLOC * 3].reshape(1, NLOC, 3)
    return energy, force, virial


# speedup vs baseline: 3.5897x; 1.9037x over previous
import functools

import jax
import jax.numpy as jnp
from jax import lax
from jax.experimental import pallas as pl
from jax.experimental.pallas import tpu as pltpu
from jax.experimental.pallas import tpu_sc as plsc

NF = 1
NLOC = 10000
NALL = 12000
NNEI = 32
M = 64
AXIS = 8
NTYPES = 2

B = 64
P = B * NNEI
NLOCP = 10240
NBLK = NLOCP // B
NPF = NLOCP * NNEI
NP = NLOC * NNEI

TILES = 32
APT = NLOCP // TILES
FPAD = 30720
SL = FPAD // TILES
SH = 1200


def _tc_body(rx_r, ry_r, rz_r, at_r, W0T_r, b0c_r, W1_r, W1T_r, b1c_r, W2_r,
             W2T_r, b2c_r, FW0r_r, FW0rT_r, Fb0c_r, FW1_r, FW1T_r, Fb1c_r,
             FW2c_r, bias_r, Seg_r, E_r, gx_r, gy_r, gz_r, cs_r, acc_r):
    i = pl.program_id(0)
    f32 = jnp.float32

    pair_atom = jax.lax.broadcasted_iota(jnp.int32, (1, P), 1) // NNEI
    pmask = (pair_atom + i * B) < NLOC
    amask = (jax.lax.broadcasted_iota(jnp.int32, (1, B), 1) + i * B) < NLOC

    rx = jnp.where(pmask, rx_r[0], 1.0)
    ry = jnp.where(pmask, ry_r[0], 1.0)
    rz = jnp.where(pmask, rz_r[0], 1.0)

    rr = rx * rx + ry * ry + rz * rz + 1e-6
    inv_rr = 1.0 / rr
    s = 1.0 / jnp.sqrt(rr)


    def bl(x, rows):
        return jnp.broadcast_to(x, (rows, x.shape[1]))

    def bc(col, cols):
        return jnp.broadcast_to(col, (col.shape[0], cols))

    h0 = jnp.tanh(bc(W0T_r[...], P) * bl(s, 16) + bc(b0c_r[...], P))
    h1 = jnp.tanh(jnp.dot(W1T_r[...], h0, preferred_element_type=f32, precision=lax.Precision.DEFAULT)
                  + bc(b1c_r[...], P))
    g = jnp.tanh(jnp.dot(W2T_r[...], h1, preferred_element_type=f32, precision=lax.Precision.DEFAULT)
                 + bc(b2c_r[...], P))

    env = (s, rx / rr, ry / rr, rz / rr)

    Seg = Seg_r[...]
    E = E_r[...]

    GR = [jnp.dot(g * bl(env[c], M), Seg, preferred_element_type=f32, precision=lax.Precision.DEFAULT)
          for c in range(4)]

    accf = bc(Fb0c_r[...], B)
    for a in range(AXIS):
        d_a = GR[0] * bl(GR[0][a:a + 1, :], M)
        for c in range(1, 4):
            d_a = d_a + GR[c] * bl(GR[c][a:a + 1, :], M)
        accf = accf + jnp.dot(FW0rT_r[a], d_a, preferred_element_type=f32, precision=lax.Precision.DEFAULT)
    h0f = jnp.tanh(accf)
    h1f = jnp.tanh(jnp.dot(FW1T_r[...], h0f, preferred_element_type=f32, precision=lax.Precision.DEFAULT)
                   + bc(Fb1c_r[...], B))
    fw2b = bc(FW2c_r[...], B)
    aepre = jnp.sum(h1f * fw2b, axis=0, keepdims=True)
    bias0 = bias_r[0, 0]
    bias1 = bias_r[0, 1]
    atv = at_r[0]
    ae = aepre + jnp.where(atv == 0, bias0, bias1)
    esum = jnp.sum(jnp.where(amask, ae, 0.0))

    u1 = (1.0 - h1f * h1f) * fw2b
    u0 = jnp.dot(FW1_r[...], u1, preferred_element_type=f32, precision=lax.Precision.DEFAULT) * (1.0 - h0f * h0f)
    dGR = []
    dDa = [jnp.dot(FW0r_r[a], u0, preferred_element_type=f32, precision=lax.Precision.DEFAULT)
           for a in range(AXIS)]
    for c in range(4):
        t = dDa[0] * bl(GR[c][0:1, :], M)
        for a in range(1, AXIS):
            t = t + dDa[a] * bl(GR[c][a:a + 1, :], M)
        rows = [jnp.sum(dDa[m] * GR[c], axis=0, keepdims=True)
                for m in range(AXIS)]
        t2 = jnp.concatenate(rows + [jnp.zeros((M - AXIS, B), f32)], axis=0)
        dGR.append(t + t2)

    dGRe = [jnp.dot(dGR[c], E, preferred_element_type=f32, precision=lax.Precision.DEFAULT) for c in range(4)]
    dg = dGRe[0] * bl(env[0], M)
    for c in range(1, 4):
        dg = dg + dGRe[c] * bl(env[c], M)
    denv = [jnp.sum(dGRe[c] * g, axis=0, keepdims=True)
            for c in range(4)]

    t2e = dg * (1.0 - g * g)
    dh1e = jnp.dot(W2_r[...], t2e, preferred_element_type=f32, precision=lax.Precision.DEFAULT) * (1.0 - h1 * h1)
    dh0e = jnp.dot(W1_r[...], dh1e, preferred_element_type=f32, precision=lax.Precision.DEFAULT) * (1.0 - h0 * h0)
    ds_embed = jnp.sum(dh0e * bc(W0T_r[...], P), axis=0, keepdims=True)

    ds_total = denv[0] + ds_embed
    dot = denv[1] * rx + denv[2] * ry + denv[3] * rz
    s3 = s * inv_rr
    common = 2.0 * dot * inv_rr * inv_rr
    gx = denv[1] * inv_rr - rx * common - ds_total * rx * s3
    gy = denv[2] * inv_rr - ry * common - ds_total * ry * s3
    gz = denv[3] * inv_rr - rz * common - ds_total * rz * s3
    gx = jnp.where(pmask, gx, 0.0)
    gy = jnp.where(pmask, gy, 0.0)
    gz = jnp.where(pmask, gz, 0.0)
    gx_r[...] = gx[None]
    gy_r[...] = gy[None]
    gz_r[...] = gz[None]

    csx = jnp.dot(gx, Seg, preferred_element_type=f32, precision=lax.Precision.HIGHEST) * f32(NNEI)
    csy = jnp.dot(gy, Seg, preferred_element_type=f32, precision=lax.Precision.HIGHEST) * f32(NNEI)
    csz = jnp.dot(gz, Seg, preferred_element_type=f32, precision=lax.Precision.HIGHEST) * f32(NNEI)
    cs_r[...] = jnp.concatenate([csx, csy, csz], axis=1)[None]

    parts = [esum]
    for rc in (rx, ry, rz):
        for gc in (gx, gy, gz):
            parts.append(-jnp.sum(rc * gc))
    pvec = jnp.concatenate([x.reshape(1, 1) for x in parts], axis=1)
    pvec = jnp.concatenate([pvec, jnp.zeros((1, 128 - len(parts)), f32)],
                           axis=1)

    @pl.when(i == 0)
    def _():
        acc_r[...] = jnp.zeros_like(acc_r)

    acc_r[...] += pvec


def _tc_call(rx, ry, rz, atype_p, W0T, b0c, W1, W1T, b1c, W2, W2T, b2c, FW0r,
             FW0rT, Fb0c, FW1, FW1T, Fb1c, FW2c, biasadj, Seg, E):
    f32 = jnp.float32
    full = lambda shp: pl.BlockSpec(shp, lambda i: (0,) * len(shp))
    row = pl.BlockSpec((1, 1, P), lambda i: (i, 0, 0))
    out = pl.pallas_call(
        _tc_body,
        grid=(NBLK,),
        in_specs=[
            row, row, row,
            pl.BlockSpec((1, 1, B), lambda i: (i, 0, 0)),
            full((16, 1)), full((16, 1)), full((16, 32)), full((32, 16)),
            full((32, 1)), full((32, 64)), full((64, 32)), full((64, 1)),
            full((8, 64, 128)), full((8, 128, 64)), full((128, 1)),
            full((128, 128)), full((128, 128)), full((128, 1)),
            full((128, 1)), full((1, NTYPES)),
            full((P, B)), full((B, P)),
        ],
        out_specs=[
            row, row, row,
            pl.BlockSpec((1, 1, 192), lambda i: (i, 0, 0)),
            pl.BlockSpec((1, 128), lambda i: (0, 0)),
        ],
        out_shape=[
            jax.ShapeDtypeStruct((NBLK, 1, P), f32),
            jax.ShapeDtypeStruct((NBLK, 1, P), f32),
            jax.ShapeDtypeStruct((NBLK, 1, P), f32),
            jax.ShapeDtypeStruct((NBLK, 1, 192), f32),
            jax.ShapeDtypeStruct((1, 128), f32),
        ],
        compiler_params=pltpu.CompilerParams(
            dimension_semantics=("arbitrary",)),
    )(rx, ry, rz, atype_p, W0T, b0c, W1, W1T, b1c, W2, W2T, b2c, FW0r, FW0rT,
      Fb0c, FW1, FW1T, Fb1c, FW2c, biasadj, Seg, E)
    return out


def _sc_gather_body(cx_h, cy_h, cz_h, map_h, shx_h, shy_h, shz_h, sel_h,
                    rx_h, ry_h, rz_h,
                    cx, cy, cz, mp, ecx, ecy, ecz, sx, sy, sz, selv,
                    ox, oy, oz):
    info = plsc.get_sparse_core_info()
    wid = lax.axis_index("s") * info.num_cores + lax.axis_index("c")

    pltpu.sync_copy(cx_h, cx)
    pltpu.sync_copy(cy_h, cy)
    pltpu.sync_copy(cz_h, cz)
    pltpu.sync_copy(map_h, mp)

    def chunk_body(ci, carry):
        base = ci * SH
        pltpu.sync_copy(shx_h.at[pl.ds(base, SH)], sx)
        pltpu.sync_copy(shy_h.at[pl.ds(base, SH)], sy)
        pltpu.sync_copy(shz_h.at[pl.ds(base, SH)], sz)

        def vec_body(vi, c2):
            o = vi * 16
            idx = mp[pl.ds(base + o, 16)]
            ecx[pl.ds(base + o, 16)] = plsc.load_gather(cx, [idx]) - sx[pl.ds(o, 16)]
            ecy[pl.ds(base + o, 16)] = plsc.load_gather(cy, [idx]) - sy[pl.ds(o, 16)]
            ecz[pl.ds(base + o, 16)] = plsc.load_gather(cz, [idx]) - sz[pl.ds(o, 16)]
            return c2

        return lax.fori_loop(0, SH // 16, vec_body, carry)

    lax.fori_loop(0, NALL // SH, chunk_body, 0)

    a0 = wid * APT
    pltpu.sync_copy(sel_h.at[pl.ds(a0 * NNEI, APT * NNEI)], selv)

    CH = 16

    def out_chunk(oi, carry):
        def atom_body(ai, c2):
            i = oi * CH + ai
            n = a0 + i
            nv = jnp.full((16,), n, jnp.int32)
            cx0 = plsc.load_gather(ecx, [nv])
            cy0 = plsc.load_gather(ecy, [nv])
            cz0 = plsc.load_gather(ecz, [nv])
            for u in range(NNEI // 16):
                jo = i * NNEI + u * 16
                so = ai * NNEI + u * 16
                jdx = selv[pl.ds(jo, 16)]
                ox[pl.ds(so, 16)] = plsc.load_gather(ecx, [jdx]) - cx0
                oy[pl.ds(so, 16)] = plsc.load_gather(ecy, [jdx]) - cy0
                oz[pl.ds(so, 16)] = plsc.load_gather(ecz, [jdx]) - cz0
            return c2

        lax.fori_loop(0, CH, atom_body, 0)
        dst = pl.ds((a0 + oi * CH) * NNEI, CH * NNEI)
        pltpu.sync_copy(ox, rx_h.at[dst])
        pltpu.sync_copy(oy, ry_h.at[dst])
        pltpu.sync_copy(oz, rz_h.at[dst])
        return carry

    lax.fori_loop(0, APT // CH, out_chunk, 0)


def _sc_gather(cx, cy, cz, mapping, shx, shy, shz, sel_pad):
    f32 = jnp.float32
    mesh = plsc.VectorSubcoreMesh(core_axis_name="c", subcore_axis_name="s")
    kern = pl.kernel(
        _sc_gather_body,
        out_type=[jax.ShapeDtypeStruct((NPF,), f32)] * 3,
        mesh=mesh,
        scratch_types=[
            pltpu.VMEM((NLOC,), f32),
            pltpu.VMEM((NLOC,), f32),
            pltpu.VMEM((NLOC,), f32),
            pltpu.VMEM((NALL,), jnp.int32),
            pltpu.VMEM((NALL,), f32),
            pltpu.VMEM((NALL,), f32),
            pltpu.VMEM((NALL,), f32),
            pltpu.VMEM((SH,), f32),
            pltpu.VMEM((SH,), f32),
            pltpu.VMEM((SH,), f32),
            pltpu.VMEM((APT * NNEI,), jnp.int32),
            pltpu.VMEM((16 * NNEI,), f32),
            pltpu.VMEM((16 * NNEI,), f32),
            pltpu.VMEM((16 * NNEI,), f32),
        ],
        compiler_params=pltpu.CompilerParams(needs_layout_passes=False),
    )
    return kern(cx, cy, cz, mapping, shx, shy, shz, sel_pad)


def _sc_scatter_body(gx_h, gy_h, gz_h, cs_h, map_h, sel_h, part_h,
                     facc, mp, selv, gxv, gyv, gzv, csv):
    info = plsc.get_sparse_core_info()
    wid = lax.axis_index("s") * info.num_cores + lax.axis_index("c")
    a0 = wid * APT

    pltpu.sync_copy(map_h, mp)
    pltpu.sync_copy(sel_h.at[pl.ds(a0 * NNEI, APT * NNEI)], selv)
    pltpu.sync_copy(gx_h.at[pl.ds(a0 * NNEI, APT * NNEI)], gxv)
    pltpu.sync_copy(gy_h.at[pl.ds(a0 * NNEI, APT * NNEI)], gyv)
    pltpu.sync_copy(gz_h.at[pl.ds(a0 * NNEI, APT * NNEI)], gzv)
    pltpu.sync_copy(cs_h.at[pl.ds((a0 // B) * 192, (APT // B) * 192)], csv)

    zero16 = jnp.zeros((16,), jnp.float32)

    def zero_body(i, c):
        facc[pl.ds(i * 16, 16)] = zero16
        return c

    lax.fori_loop(0, FPAD // 16, zero_body, 0)

    nvalid = jnp.maximum(0, jnp.minimum(APT, NLOC - a0))

    def scatter3(midx, vx, vy, vz):
        cnt, _ = plsc.scan_count(midx)
        maxc = jnp.max(cnt, axis=0)
        t = midx * 3

        def round_body(r, c):
            m = cnt == r
            plsc.addupdate_scatter(facc, [t], vx, mask=m)
            plsc.addupdate_scatter(facc, [t + 1], vy, mask=m)
            plsc.addupdate_scatter(facc, [t + 2], vz, mask=m)
            return c

        return lax.fori_loop(1, maxc + 1, round_body, 0)

    def atom_body(i, c):
        for u in range(NNEI // 16):
            off = i * NNEI + u * 16
            jdx = selv[pl.ds(off, 16)]
            msel = plsc.load_gather(mp, [jdx])
            scatter3(msel, -gxv[pl.ds(off, 16)], -gyv[pl.ds(off, 16)],
                     -gzv[pl.ds(off, 16)])
        return c

    lax.fori_loop(0, nvalid, atom_body, 0)

    def cent_body(k, c):
        i = k * 16
        mvec = mp[pl.ds(a0 + i, 16)]
        blk = (i // B) * 192
        io = i % B
        scatter3(mvec, csv[pl.ds(blk + io, 16)], csv[pl.ds(blk + B + io, 16)],
                 csv[pl.ds(blk + 2 * B + io, 16)])
        return c

    lax.fori_loop(0, nvalid // 16, cent_body, 0)

    pltpu.sync_copy(facc, part_h.at[pl.ds(wid * FPAD, FPAD)])


def _sc_scatter(gx, gy, gz, cs, mapping, sel_pad):
    f32 = jnp.float32
    mesh = plsc.VectorSubcoreMesh(core_axis_name="c", subcore_axis_name="s")
    kern = pl.kernel(
        _sc_scatter_body,
        out_type=jax.ShapeDtypeStruct((TILES * FPAD,), f32),
        mesh=mesh,
        scratch_types=[
            pltpu.VMEM((FPAD,), f32),
            pltpu.VMEM((NALL,), jnp.int32),
            pltpu.VMEM((APT * NNEI,), jnp.int32),
            pltpu.VMEM((APT * NNEI,), f32),
            pltpu.VMEM((APT * NNEI,), f32),
            pltpu.VMEM((APT * NNEI,), f32),
            pltpu.VMEM(((APT // B) * 192,), f32),
        ],
        compiler_params=pltpu.CompilerParams(needs_layout_passes=False),
    )
    return kern(gx, gy, gz, cs, mapping, sel_pad)


def _sc_reduce_body(part_h, force_h, pbuf, obuf):
    info = plsc.get_sparse_core_info()
    wid = lax.axis_index("s") * info.num_cores + lax.axis_index("c")
    base = wid * SL
    for t in range(TILES):
        pltpu.sync_copy(part_h.at[pl.ds(t * FPAD + base, SL)],
                        pbuf.at[pl.ds(t * SL, SL)])

    def vec_body(v, c):
        o = v * 16
        acc = pbuf[pl.ds(o, 16)]
        for t in range(1, TILES):
            acc = acc + pbuf[pl.ds(t * SL + o, 16)]
        obuf[pl.ds(o, 16)] = acc
        return c

    lax.fori_loop(0, SL // 16, vec_body, 0)
    pltpu.sync_copy(obuf, force_h.at[pl.ds(base, SL)])


def _sc_reduce(partials):
    f32 = jnp.float32
    mesh = plsc.VectorSubcoreMesh(core_axis_name="c", subcore_axis_name="s")
    kern = pl.kernel(
        _sc_reduce_body,
        out_type=jax.ShapeDtypeStruct((FPAD,), f32),
        mesh=mesh,
        scratch_types=[
            pltpu.VMEM((TILES * SL,), f32),
            pltpu.VMEM((SL,), f32),
        ],
        compiler_params=pltpu.CompilerParams(needs_layout_passes=False),
    )
    return kern(partials)


def kernel(coord, atype, natoms, mapping, shift, selected, box, W0, b0, W1,
           b1, W2, b2, FW0, Fb0, FW1, Fb1, FW2, Fb2, bias_atom_e):
    f32 = jnp.float32
    cx = coord[0, :, 0]
    cy = coord[0, :, 1]
    cz = coord[0, :, 2]
    shx = shift[0, :, 0]
    shy = shift[0, :, 1]
    shz = shift[0, :, 2]
    map0 = mapping[0].astype(jnp.int32)
    sel_pad = jnp.concatenate(
        [selected[0].reshape(NP).astype(jnp.int32),
         jnp.zeros((NPF - NP,), jnp.int32)])

    rx, ry, rz = _sc_gather(cx, cy, cz, map0, shx, shy, shz, sel_pad)
    rx = rx.reshape(NBLK, 1, P)
    ry = ry.reshape(NBLK, 1, P)
    rz = rz.reshape(NBLK, 1, P)

    atype_p = jnp.concatenate(
        [atype[0].astype(jnp.int32), jnp.zeros((NLOCP - NLOC,), jnp.int32)]
    ).reshape(NBLK, 1, B)
    W0T = W0.T
    b0c = b0.reshape(16, 1)
    W1T = W1.T
    b1c = b1.reshape(32, 1)
    W2T = W2.T
    b2c = b2.reshape(M, 1)
    FW0r = FW0.reshape(M, AXIS, 128).transpose(1, 0, 2)
    FW0rT = FW0r.transpose(0, 2, 1)
    Fb0c = Fb0.reshape(128, 1)
    FW1T = FW1.T
    Fb1c = Fb1.reshape(128, 1)
    biasadj = (bias_atom_e + Fb2[0]).reshape(1, NTYPES)
    pa = jnp.arange(P, dtype=jnp.int32)[:, None] // NNEI
    nb = jnp.arange(B, dtype=jnp.int32)[None, :]
    Seg = jnp.where(pa == nb, jnp.float32(1.0 / NNEI), 0.0)
    E = Seg.T

    gx, gy, gz, cs, acc = _tc_call(rx, ry, rz, atype_p, W0T, b0c, W1, W1T,
                                   b1c, W2, W2T, b2c, FW0r, FW0rT, Fb0c, FW1,
                                   FW1T, Fb1c, FW2, biasadj, Seg, E)

    partials = _sc_scatter(gx.reshape(NPF), gy.reshape(NPF), gz.reshape(NPF),
                           cs.reshape(NBLK * 192), map0, sel_pad)
    force_flat = _sc_reduce(partials)

    energy = acc[0, 0:1]
    virial = acc[0, 1:10].reshape(1, 3, 3)
    force = force_flat[:NLOC * 3].reshape(1, NLOC, 3)
    return energy, force, virial


# speedup vs baseline: 3.5968x; 1.0020x over previous
import jax
import jax.numpy as jnp
from jax import lax
from jax.experimental import pallas as pl
from jax.experimental.pallas import tpu as pltpu
from jax.experimental.pallas import tpu_sc as plsc

NF = 1
NLOC = 10000
NALL = 12000
NNEI = 32
M = 64
AXIS = 8
NTYPES = 2

B = 64
P = B * NNEI
NLOCP = 10240
NBLK = NLOCP // B
NPF = NLOCP * NNEI
NP = NLOC * NNEI

TILES = 32
APT = NLOCP // TILES
FPAD = 30720
SL = FPAD // TILES
SH = 1200


def _tc_body(rx_r, ry_r, rz_r, at_r, W0T_r, b0c_r, W1_r, W1T_r, b1c_r, W2_r,
             W2T_r, b2c_r, FW0r_r, FW0rT_r, Fb0c_r, FW1_r, FW1T_r, Fb1c_r,
             FW2c_r, bias_r, Seg_r, E_r, gx_r, gy_r, gz_r, cs_r, acc_r):
    i = pl.program_id(0)
    f32 = jnp.float32

    pair_atom = jax.lax.broadcasted_iota(jnp.int32, (1, P), 1) // NNEI
    pmask = (pair_atom + i * B) < NLOC
    amask = (jax.lax.broadcasted_iota(jnp.int32, (1, B), 1) + i * B) < NLOC

    rx = jnp.where(pmask, rx_r[0], 1.0)
    ry = jnp.where(pmask, ry_r[0], 1.0)
    rz = jnp.where(pmask, rz_r[0], 1.0)

    rr = rx * rx + ry * ry + rz * rz + 1e-6
    inv_rr = 1.0 / rr
    s = 1.0 / jnp.sqrt(rr)


    def bl(x, rows):
        return jnp.broadcast_to(x, (rows, x.shape[1]))

    def bc(col, cols):
        return jnp.broadcast_to(col, (col.shape[0], cols))

    h0 = jnp.tanh(bc(W0T_r[...], P) * bl(s, 16) + bc(b0c_r[...], P))
    h1 = jnp.tanh(jnp.dot(W1T_r[...], h0, preferred_element_type=f32, precision=lax.Precision.DEFAULT)
                  + bc(b1c_r[...], P))
    g = jnp.tanh(jnp.dot(W2T_r[...], h1, preferred_element_type=f32, precision=lax.Precision.DEFAULT)
                 + bc(b2c_r[...], P))

    env = (s, rx / rr, ry / rr, rz / rr)

    Seg = Seg_r[...]
    E = E_r[...]

    GR = [jnp.dot(g * bl(env[c], M), Seg, preferred_element_type=f32, precision=lax.Precision.DEFAULT)
          for c in range(4)]

    accf = bc(Fb0c_r[...], B)
    for a in range(AXIS):
        d_a = GR[0] * bl(GR[0][a:a + 1, :], M)
        for c in range(1, 4):
            d_a = d_a + GR[c] * bl(GR[c][a:a + 1, :], M)
        accf = accf + jnp.dot(FW0rT_r[a], d_a, preferred_element_type=f32, precision=lax.Precision.DEFAULT)
    h0f = jnp.tanh(accf)
    h1f = jnp.tanh(jnp.dot(FW1T_r[...], h0f, preferred_element_type=f32, precision=lax.Precision.DEFAULT)
                   + bc(Fb1c_r[...], B))
    fw2b = bc(FW2c_r[...], B)
    aepre = jnp.sum(h1f * fw2b, axis=0, keepdims=True)
    bias0 = bias_r[0, 0]
    bias1 = bias_r[0, 1]
    atv = at_r[0]
    ae = aepre + jnp.where(atv == 0, bias0, bias1)
    esum = jnp.sum(jnp.where(amask, ae, 0.0))

    u1 = (1.0 - h1f * h1f) * fw2b
    u0 = jnp.dot(FW1_r[...], u1, preferred_element_type=f32, precision=lax.Precision.DEFAULT) * (1.0 - h0f * h0f)
    dGR = []
    dDa = [jnp.dot(FW0r_r[a], u0, preferred_element_type=f32, precision=lax.Precision.DEFAULT)
           for a in range(AXIS)]
    for c in range(4):
        t = dDa[0] * bl(GR[c][0:1, :], M)
        for a in range(1, AXIS):
            t = t + dDa[a] * bl(GR[c][a:a + 1, :], M)
        rows = [jnp.sum(dDa[m] * GR[c], axis=0, keepdims=True)
                for m in range(AXIS)]
        t2 = jnp.concatenate(rows + [jnp.zeros((M - AXIS, B), f32)], axis=0)
        dGR.append(t + t2)

    dGRe = [jnp.dot(dGR[c], E, preferred_element_type=f32, precision=lax.Precision.DEFAULT) for c in range(4)]
    dg = dGRe[0] * bl(env[0], M)
    for c in range(1, 4):
        dg = dg + dGRe[c] * bl(env[c], M)
    denv = [jnp.sum(dGRe[c] * g, axis=0, keepdims=True)
            for c in range(4)]

    t2e = dg * (1.0 - g * g)
    dh1e = jnp.dot(W2_r[...], t2e, preferred_element_type=f32, precision=lax.Precision.DEFAULT) * (1.0 - h1 * h1)
    dh0e = jnp.dot(W1_r[...], dh1e, preferred_element_type=f32, precision=lax.Precision.DEFAULT) * (1.0 - h0 * h0)
    ds_embed = jnp.sum(dh0e * bc(W0T_r[...], P), axis=0, keepdims=True)

    ds_total = denv[0] + ds_embed
    dot = denv[1] * rx + denv[2] * ry + denv[3] * rz
    s3 = s * inv_rr
    common = 2.0 * dot * inv_rr * inv_rr
    gx = denv[1] * inv_rr - rx * common - ds_total * rx * s3
    gy = denv[2] * inv_rr - ry * common - ds_total * ry * s3
    gz = denv[3] * inv_rr - rz * common - ds_total * rz * s3
    gx = jnp.where(pmask, gx, 0.0)
    gy = jnp.where(pmask, gy, 0.0)
    gz = jnp.where(pmask, gz, 0.0)
    gx_r[...] = gx[None]
    gy_r[...] = gy[None]
    gz_r[...] = gz[None]

    csx = jnp.dot(gx, Seg, preferred_element_type=f32, precision=lax.Precision.HIGHEST) * f32(NNEI)
    csy = jnp.dot(gy, Seg, preferred_element_type=f32, precision=lax.Precision.HIGHEST) * f32(NNEI)
    csz = jnp.dot(gz, Seg, preferred_element_type=f32, precision=lax.Precision.HIGHEST) * f32(NNEI)
    cs_r[...] = jnp.concatenate([csx, csy, csz], axis=1)[None]

    parts = [esum]
    for rc in (rx, ry, rz):
        for gc in (gx, gy, gz):
            parts.append(-jnp.sum(rc * gc))
    pvec = jnp.concatenate([x.reshape(1, 1) for x in parts], axis=1)
    pvec = jnp.concatenate([pvec, jnp.zeros((1, 128 - len(parts)), f32)],
                           axis=1)

    @pl.when(i == 0)
    def _():
        acc_r[...] = jnp.zeros_like(acc_r)

    acc_r[...] += pvec


def _tc_call(rx, ry, rz, atype_p, W0T, b0c, W1, W1T, b1c, W2, W2T, b2c, FW0r,
             FW0rT, Fb0c, FW1, FW1T, Fb1c, FW2c, biasadj, Seg, E):
    f32 = jnp.float32
    full = lambda shp: pl.BlockSpec(shp, lambda i: (0,) * len(shp))
    row = pl.BlockSpec((1, 1, P), lambda i: (i, 0, 0))
    out = pl.pallas_call(
        _tc_body,
        grid=(NBLK,),
        in_specs=[
            row, row, row,
            pl.BlockSpec((1, 1, B), lambda i: (i, 0, 0)),
            full((16, 1)), full((16, 1)), full((16, 32)), full((32, 16)),
            full((32, 1)), full((32, 64)), full((64, 32)), full((64, 1)),
            full((8, 64, 128)), full((8, 128, 64)), full((128, 1)),
            full((128, 128)), full((128, 128)), full((128, 1)),
            full((128, 1)), full((1, NTYPES)),
            full((P, B)), full((B, P)),
        ],
        out_specs=[
            row, row, row,
            pl.BlockSpec((1, 1, 192), lambda i: (i, 0, 0)),
            pl.BlockSpec((1, 128), lambda i: (0, 0)),
        ],
        out_shape=[
            jax.ShapeDtypeStruct((NBLK, 1, P), f32),
            jax.ShapeDtypeStruct((NBLK, 1, P), f32),
            jax.ShapeDtypeStruct((NBLK, 1, P), f32),
            jax.ShapeDtypeStruct((NBLK, 1, 192), f32),
            jax.ShapeDtypeStruct((1, 128), f32),
        ],
        compiler_params=pltpu.CompilerParams(
            dimension_semantics=("arbitrary",)),
    )(rx, ry, rz, atype_p, W0T, b0c, W1, W1T, b1c, W2, W2T, b2c, FW0r, FW0rT,
      Fb0c, FW1, FW1T, Fb1c, FW2c, biasadj, Seg, E)
    return out


def _sc_gather_body(cx_h, cy_h, cz_h, map_h, shx_h, shy_h, shz_h, sel_h,
                    rx_h, ry_h, rz_h,
                    cx, cy, cz, mp, ecx, ecy, ecz, sx, sy, sz, selv,
                    ox, oy, oz):
    info = plsc.get_sparse_core_info()
    wid = lax.axis_index("s") * info.num_cores + lax.axis_index("c")

    pltpu.sync_copy(cx_h, cx)
    pltpu.sync_copy(cy_h, cy)
    pltpu.sync_copy(cz_h, cz)
    pltpu.sync_copy(map_h, mp)

    def chunk_body(ci, carry):
        base = ci * SH
        pltpu.sync_copy(shx_h.at[pl.ds(base, SH)], sx)
        pltpu.sync_copy(shy_h.at[pl.ds(base, SH)], sy)
        pltpu.sync_copy(shz_h.at[pl.ds(base, SH)], sz)

        def vec_body(vi, c2):
            o = vi * 16
            idx = mp[pl.ds(base + o, 16)]
            ecx[pl.ds(base + o, 16)] = plsc.load_gather(cx, [idx]) - sx[pl.ds(o, 16)]
            ecy[pl.ds(base + o, 16)] = plsc.load_gather(cy, [idx]) - sy[pl.ds(o, 16)]
            ecz[pl.ds(base + o, 16)] = plsc.load_gather(cz, [idx]) - sz[pl.ds(o, 16)]
            return c2

        return lax.fori_loop(0, SH // 16, vec_body, carry)

    lax.fori_loop(0, NALL // SH, chunk_body, 0)

    a0 = wid * APT
    pltpu.sync_copy(sel_h.at[pl.ds(a0 * NNEI, APT * NNEI)], selv)

    CH = 16

    def out_chunk(oi, carry):
        def atom_body(ai, c2):
            i = oi * CH + ai
            n = a0 + i
            nv = jnp.full((16,), n, jnp.int32)
            cx0 = plsc.load_gather(ecx, [nv])
            cy0 = plsc.load_gather(ecy, [nv])
            cz0 = plsc.load_gather(ecz, [nv])
            for u in range(NNEI // 16):
                jo = i * NNEI + u * 16
                so = ai * NNEI + u * 16
                jdx = selv[pl.ds(jo, 16)]
                ox[pl.ds(so, 16)] = plsc.load_gather(ecx, [jdx]) - cx0
                oy[pl.ds(so, 16)] = plsc.load_gather(ecy, [jdx]) - cy0
                oz[pl.ds(so, 16)] = plsc.load_gather(ecz, [jdx]) - cz0
            return c2

        lax.fori_loop(0, CH, atom_body, 0)
        dst = pl.ds((a0 + oi * CH) * NNEI, CH * NNEI)
        pltpu.sync_copy(ox, rx_h.at[dst])
        pltpu.sync_copy(oy, ry_h.at[dst])
        pltpu.sync_copy(oz, rz_h.at[dst])
        return carry

    lax.fori_loop(0, APT // CH, out_chunk, 0)


def _sc_gather(cx, cy, cz, mapping, shx, shy, shz, sel_pad):
    f32 = jnp.float32
    mesh = plsc.VectorSubcoreMesh(core_axis_name="c", subcore_axis_name="s")
    kern = pl.kernel(
        _sc_gather_body,
        out_type=[jax.ShapeDtypeStruct((NPF,), f32)] * 3,
        mesh=mesh,
        scratch_types=[
            pltpu.VMEM((NLOC,), f32),
            pltpu.VMEM((NLOC,), f32),
            pltpu.VMEM((NLOC,), f32),
            pltpu.VMEM((NALL,), jnp.int32),
            pltpu.VMEM((NALL,), f32),
            pltpu.VMEM((NALL,), f32),
            pltpu.VMEM((NALL,), f32),
            pltpu.VMEM((SH,), f32),
            pltpu.VMEM((SH,), f32),
            pltpu.VMEM((SH,), f32),
            pltpu.VMEM((APT * NNEI,), jnp.int32),
            pltpu.VMEM((16 * NNEI,), f32),
            pltpu.VMEM((16 * NNEI,), f32),
            pltpu.VMEM((16 * NNEI,), f32),
        ],
        compiler_params=pltpu.CompilerParams(needs_layout_passes=False),
    )
    return kern(cx, cy, cz, mapping, shx, shy, shz, sel_pad)


def _sc_scatter_body(gx_h, gy_h, gz_h, cs_h, map_h, sel_h, part_h,
                     facc, mp, selv, gxv, gyv, gzv, csv):
    info = plsc.get_sparse_core_info()
    wid = lax.axis_index("s") * info.num_cores + lax.axis_index("c")
    a0 = wid * APT

    pltpu.sync_copy(map_h, mp)
    pltpu.sync_copy(sel_h.at[pl.ds(a0 * NNEI, APT * NNEI)], selv)
    pltpu.sync_copy(gx_h.at[pl.ds(a0 * NNEI, APT * NNEI)], gxv)
    pltpu.sync_copy(gy_h.at[pl.ds(a0 * NNEI, APT * NNEI)], gyv)
    pltpu.sync_copy(gz_h.at[pl.ds(a0 * NNEI, APT * NNEI)], gzv)
    pltpu.sync_copy(cs_h.at[pl.ds((a0 // B) * 192, (APT // B) * 192)], csv)

    zero16 = jnp.zeros((16,), jnp.float32)

    def zero_body(i, c):
        facc[pl.ds(i * 16, 16)] = zero16
        return c

    lax.fori_loop(0, FPAD // 16, zero_body, 0)

    nvalid = jnp.maximum(0, jnp.minimum(APT, NLOC - a0))

    def scatter3(midx, vx, vy, vz):
        cnt, _ = plsc.scan_count(midx)
        maxc = jnp.max(cnt, axis=0)
        t = midx * 3

        def round_body(r, c):
            m = cnt == r
            plsc.addupdate_scatter(facc, [t], vx, mask=m)
            plsc.addupdate_scatter(facc, [t + 1], vy, mask=m)
            plsc.addupdate_scatter(facc, [t + 2], vz, mask=m)
            return c

        return lax.fori_loop(1, maxc + 1, round_body, 0)

    def atom_body(i, c):
        for u in range(NNEI // 16):
            off = i * NNEI + u * 16
            jdx = selv[pl.ds(off, 16)]
            msel = plsc.load_gather(mp, [jdx])
            scatter3(msel, -gxv[pl.ds(off, 16)], -gyv[pl.ds(off, 16)],
                     -gzv[pl.ds(off, 16)])
        return c

    lax.fori_loop(0, nvalid, atom_body, 0)

    def cent_body(k, c):
        i = k * 16
        mvec = mp[pl.ds(a0 + i, 16)]
        blk = (i // B) * 192
        io = i % B
        scatter3(mvec, csv[pl.ds(blk + io, 16)], csv[pl.ds(blk + B + io, 16)],
                 csv[pl.ds(blk + 2 * B + io, 16)])
        return c

    lax.fori_loop(0, nvalid // 16, cent_body, 0)

    pltpu.sync_copy(facc, part_h.at[pl.ds(wid * FPAD, FPAD)])


def _sc_scatter(gx, gy, gz, cs, mapping, sel_pad):
    f32 = jnp.float32
    mesh = plsc.VectorSubcoreMesh(core_axis_name="c", subcore_axis_name="s")
    kern = pl.kernel(
        _sc_scatter_body,
        out_type=jax.ShapeDtypeStruct((TILES * FPAD,), f32),
        mesh=mesh,
        scratch_types=[
            pltpu.VMEM((FPAD,), f32),
            pltpu.VMEM((NALL,), jnp.int32),
            pltpu.VMEM((APT * NNEI,), jnp.int32),
            pltpu.VMEM((APT * NNEI,), f32),
            pltpu.VMEM((APT * NNEI,), f32),
            pltpu.VMEM((APT * NNEI,), f32),
            pltpu.VMEM(((APT // B) * 192,), f32),
        ],
        compiler_params=pltpu.CompilerParams(needs_layout_passes=False),
    )
    return kern(gx, gy, gz, cs, mapping, sel_pad)


def _sc_reduce_body(part_h, force_h, pbuf, obuf):
    info = plsc.get_sparse_core_info()
    wid = lax.axis_index("s") * info.num_cores + lax.axis_index("c")
    base = wid * SL
    for t in range(TILES):
        pltpu.sync_copy(part_h.at[pl.ds(t * FPAD + base, SL)],
                        pbuf.at[pl.ds(t * SL, SL)])

    def vec_body(v, c):
        o = v * 16
        acc = pbuf[pl.ds(o, 16)]
        for t in range(1, TILES):
            acc = acc + pbuf[pl.ds(t * SL + o, 16)]
        obuf[pl.ds(o, 16)] = acc
        return c

    lax.fori_loop(0, SL // 16, vec_body, 0)
    pltpu.sync_copy(obuf, force_h.at[pl.ds(base, SL)])


def _sc_reduce(partials):
    f32 = jnp.float32
    mesh = plsc.VectorSubcoreMesh(core_axis_name="c", subcore_axis_name="s")
    kern = pl.kernel(
        _sc_reduce_body,
        out_type=jax.ShapeDtypeStruct((FPAD,), f32),
        mesh=mesh,
        scratch_types=[
            pltpu.VMEM((TILES * SL,), f32),
            pltpu.VMEM((SL,), f32),
        ],
        compiler_params=pltpu.CompilerParams(needs_layout_passes=False),
    )
    return kern(partials)


def kernel(coord, atype, natoms, mapping, shift, selected, box, W0, b0, W1,
           b1, W2, b2, FW0, Fb0, FW1, Fb1, FW2, Fb2, bias_atom_e):
    f32 = jnp.float32
    cx = coord[0, :, 0]
    cy = coord[0, :, 1]
    cz = coord[0, :, 2]
    shx = shift[0, :, 0]
    shy = shift[0, :, 1]
    shz = shift[0, :, 2]
    map0 = mapping[0].astype(jnp.int32)
    sel_pad = jnp.concatenate(
        [selected[0].reshape(NP).astype(jnp.int32),
         jnp.zeros((NPF - NP,), jnp.int32)])

    rx, ry, rz = _sc_gather(cx, cy, cz, map0, shx, shy, shz, sel_pad)
    rx = rx.reshape(NBLK, 1, P)
    ry = ry.reshape(NBLK, 1, P)
    rz = rz.reshape(NBLK, 1, P)

    atype_p = jnp.concatenate(
        [atype[0].astype(jnp.int32), jnp.zeros((NLOCP - NLOC,), jnp.int32)]
    ).reshape(NBLK, 1, B)
    W0T = W0.T
    b0c = b0.reshape(16, 1)
    W1T = W1.T
    b1c = b1.reshape(32, 1)
    W2T = W2.T
    b2c = b2.reshape(M, 1)
    FW0r = FW0.reshape(M, AXIS, 128).transpose(1, 0, 2)
    FW0rT = FW0r.transpose(0, 2, 1)
    Fb0c = Fb0.reshape(128, 1)
    FW1T = FW1.T
    Fb1c = Fb1.reshape(128, 1)
    biasadj = (bias_atom_e + Fb2[0]).reshape(1, NTYPES)
    pa = jnp.arange(P, dtype=jnp.int32)[:, None] // NNEI
    nb = jnp.arange(B, dtype=jnp.int32)[None, :]
    Seg = jnp.where(pa == nb, jnp.float32(1.0 / NNEI), 0.0)
    E = Seg.T

    gx, gy, gz, cs, acc = _tc_call(rx, ry, rz, atype_p, W0T, b0c, W1, W1T,
                                   b1c, W2, W2T, b2c, FW0r, FW0rT, Fb0c, FW1,
                                   FW1T, Fb1c, FW2, biasadj, Seg, E)

    partials = _sc_scatter(gx.reshape(NPF), gy.reshape(NPF), gz.reshape(NPF),
                           cs.reshape(NBLK * 192), map0, sel_pad)
    force_flat = _sc_reduce(partials)

    energy = acc[0, 0:1]
    virial = acc[0, 1:10].reshape(1, 3, 3)
    force = force_flat[:NLOC * 3].reshape(1, NLOC, 3)
    return energy, force, virial
